# baseline (speedup 1.0000x reference)
import functools

import jax
import jax.numpy as jnp
from jax import lax
from jax.experimental import pallas as pl
from jax.experimental.pallas import tpu as pltpu
from jax.experimental.pallas import tpu_sc as plsc

_LANES = 16
_CHUNK = 128


def _sqrt_newton(x):
    i = lax.bitcast_convert_type(x, jnp.int32)
    i = jnp.int32(0x5F3759DF) - lax.shift_right_logical(i, 1)
    y = lax.bitcast_convert_type(i, jnp.float32)
    for _ in range(3):
        y = y * (jnp.float32(1.5) - jnp.float32(0.5) * x * y * y)
    return x * y


def _build_sc_kernel(N, L, D1, MAXPOS, rows_per_w, nc):
    mesh = plsc.VectorSubcoreMesh(core_axis_name="c", subcore_axis_name="s")
    n_chunks = rows_per_w // _CHUNK
    n_pairs = n_chunks // 2
    D = D1 - 1

    @functools.partial(
        pl.kernel,
        mesh=mesh,
        compiler_params=pltpu.CompilerParams(
            needs_layout_passes=False, use_tc_tiling_on_sc=False
        ),
        out_type=jax.ShapeDtypeStruct((N, D1), jnp.float32),
        scratch_types=[
            pltpu.VMEM((rows_per_w,), jnp.int32),
            pltpu.VMEM((_CHUNK, D), jnp.float32),
            pltpu.VMEM((_CHUNK, D), jnp.float32),
            pltpu.VMEM((_CHUNK, D1), jnp.float32),
            pltpu.VMEM((_CHUNK, D1), jnp.float32),
            pltpu.VMEM((MAXPOS, D), jnp.float32),
            pltpu.SemaphoreType.DMA,
            pltpu.SemaphoreType.DMA,
            pltpu.SemaphoreType.DMA,
            pltpu.SemaphoreType.DMA,
        ],
    )
    def sc_embed(ids_hbm, tok_hbm, pos_hbm, out_hbm,
                 idx_all, row0_v, row1_v, out0_v, out1_v, pos_v,
                 gsem0, gsem1, osem0, osem1):
        wid = lax.axis_index("s") * nc + lax.axis_index("c")
        base = wid * rows_per_w
        pltpu.sync_copy(pos_hbm, pos_v)
        pltpu.sync_copy(ids_hbm.at[pl.ds(base, rows_per_w)], idx_all)
        lanes = lax.iota(jnp.int32, _LANES)
        zeros_i = jnp.zeros((_LANES,), jnp.int32)
        rows = (row0_v, row1_v)
        outs = (out0_v, out1_v)
        gsems = (gsem0, gsem1)
        osems = (osem0, osem1)

        def gather_start(b, g):
            pltpu.make_async_copy(
                tok_hbm.at[idx_all.at[pl.ds(g * _CHUNK, _CHUNK)]],
                rows[b], gsems[b]).start()

        def gather_wait(b):
            pltpu.make_async_copy(
                tok_hbm.at[idx_all.at[pl.ds(0, _CHUNK)]],
                rows[b], gsems[b]).wait()

        def out_start(b, g):
            pltpu.make_async_copy(
                outs[b], out_hbm.at[pl.ds(base + g * _CHUNK, _CHUNK)],
                osems[b]).start()

        def out_wait(b):
            pltpu.make_async_copy(
                outs[b], out_hbm.at[pl.ds(base, _CHUNK)], osems[b]).wait()

        def compute(b, g):
            row0 = base + g * _CHUNK
            for q in range(_CHUNK // _LANES):
                rvec = q * _LANES + lanes
                pvec = lax.rem(row0 + rvec, L)
                accs = [None, None, None, None]
                for j in range(D):
                    cj = jnp.full((_LANES,), j, jnp.int32)
                    tv = plsc.load_gather(rows[b], [rvec, cj])
                    pv = plsc.load_gather(pos_v, [pvec, cj])
                    sp = tv + pv
                    plsc.store_scatter(outs[b], [rvec, cj + 1], sp)
                    sq = sp * sp
                    k = j % 4
                    accs[k] = sq if accs[k] is None else accs[k] + sq
                acc = (accs[0] + accs[1]) + (accs[2] + accs[3])
                t = _sqrt_newton(acc + jnp.float32(1.0))
                plsc.store_scatter(outs[b], [rvec, zeros_i], t)

        gather_start(0, 0)

        def pair_body(h, carry):
            for b in (0, 1):
                g = 2 * h + b

                @pl.when(g + 1 < n_chunks)
                def _():
                    gather_start(1 - b, g + 1)

                gather_wait(b)

                @pl.when(g >= 2)
                def _():
                    out_wait(b)

                compute(b, g)
                out_start(b, g)
            return carry

        lax.fori_loop(0, n_pairs, pair_body, 0)
        out_wait(0)
        out_wait(1)

    return sc_embed


def kernel(input_ids, token_table, position_table):
    B, L = input_ids.shape
    D1 = token_table.shape[-1]
    N = B * L
    info = plsc.get_sparse_core_info()
    nw = info.num_cores * info.num_subcores
    rows_per_w = N // nw
    ids = input_ids.reshape(N).astype(jnp.int32)
    tok_sp = token_table[:, 1:]
    pos_sp = position_table[:, 1:]
    sc_embed = _build_sc_kernel(
        N, L, D1, position_table.shape[0], rows_per_w, info.num_cores
    )
    out = sc_embed(ids, tok_sp, pos_sp)
    return out.reshape(B, L, D1)


# speedup vs baseline: 1.0995x; 1.0995x over previous
import functools

import jax
import jax.numpy as jnp
from jax import lax
from jax.experimental import pallas as pl
from jax.experimental.pallas import tpu as pltpu
from jax.experimental.pallas import tpu_sc as plsc

_LANES = 16
_CHUNK = 128


def _sqrt_newton(x):
    i = lax.bitcast_convert_type(x, jnp.int32)
    i = jnp.int32(0x5F3759DF) - lax.shift_right_logical(i, 1)
    y = lax.bitcast_convert_type(i, jnp.float32)
    for _ in range(3):
        y = y * (jnp.float32(1.5) - jnp.float32(0.5) * x * y * y)
    return x * y


def _build_sc_kernel(N, L, D1, MAXPOS, rows_per_w, nc):
    mesh = plsc.VectorSubcoreMesh(core_axis_name="c", subcore_axis_name="s")
    n_chunks = rows_per_w // _CHUNK
    n_pairs = n_chunks // 2
    D = D1 - 1

    @functools.partial(
        pl.kernel,
        mesh=mesh,
        compiler_params=pltpu.CompilerParams(
            needs_layout_passes=False, use_tc_tiling_on_sc=False
        ),
        out_type=jax.ShapeDtypeStruct((N, D1), jnp.float32),
        scratch_types=[
            pltpu.VMEM((rows_per_w,), jnp.int32),
            pltpu.VMEM((_CHUNK, D), jnp.float32),
            pltpu.VMEM((_CHUNK, D), jnp.float32),
            pltpu.VMEM((_CHUNK, D1), jnp.float32),
            pltpu.VMEM((_CHUNK, D1), jnp.float32),
            pltpu.VMEM((MAXPOS, D), jnp.float32),
            pltpu.SemaphoreType.DMA,
            pltpu.SemaphoreType.DMA,
            pltpu.SemaphoreType.DMA,
            pltpu.SemaphoreType.DMA,
        ],
    )
    def sc_embed(ids_hbm, tok_hbm, pos_hbm, out_hbm,
                 idx_all, row0_v, row1_v, out0_v, out1_v, pos_v,
                 gsem0, gsem1, osem0, osem1):
        wid = lax.axis_index("s") * nc + lax.axis_index("c")
        base = wid * rows_per_w
        pltpu.sync_copy(pos_hbm, pos_v)
        pltpu.sync_copy(ids_hbm.at[pl.ds(base, rows_per_w)], idx_all)
        lanes = lax.iota(jnp.int32, _LANES)
        zeros_i = jnp.zeros((_LANES,), jnp.int32)
        rows = (row0_v, row1_v)
        outs = (out0_v, out1_v)
        gsems = (gsem0, gsem1)
        osems = (osem0, osem1)

        def gather_start(b, g):
            pltpu.make_async_copy(
                tok_hbm.at[idx_all.at[pl.ds(g * _CHUNK, _CHUNK)]],
                rows[b], gsems[b]).start()

        def gather_wait(b):
            pltpu.make_async_copy(
                tok_hbm.at[idx_all.at[pl.ds(0, _CHUNK)]],
                rows[b], gsems[b]).wait()

        def out_start(b, g):
            pltpu.make_async_copy(
                outs[b], out_hbm.at[pl.ds(base + g * _CHUNK, _CHUNK)],
                osems[b]).start()

        def out_wait(b):
            pltpu.make_async_copy(
                outs[b], out_hbm.at[pl.ds(base, _CHUNK)], osems[b]).wait()

        def compute(b, g):
            row0 = base + g * _CHUNK

            def grp_body(q, c2):
                rvec = q * _LANES + lanes
                pvec = lax.rem(row0 + rvec, L)
                accs = [None, None, None, None]
                for j in range(D):
                    cj = jnp.full((_LANES,), j, jnp.int32)
                    tv = plsc.load_gather(rows[b], [rvec, cj])
                    pv = plsc.load_gather(pos_v, [pvec, cj])
                    sp = tv + pv
                    plsc.store_scatter(outs[b], [rvec, cj + 1], sp)
                    sq = sp * sp
                    k = j % 4
                    accs[k] = sq if accs[k] is None else accs[k] + sq
                acc = (accs[0] + accs[1]) + (accs[2] + accs[3])
                t = _sqrt_newton(acc + jnp.float32(1.0))
                plsc.store_scatter(outs[b], [rvec, zeros_i], t)
                return c2

            lax.fori_loop(0, _CHUNK // _LANES, grp_body, 0)

        gather_start(0, 0)

        def pair_body(h, carry):
            for b in (0, 1):
                g = 2 * h + b

                @pl.when(g + 1 < n_chunks)
                def _():
                    gather_start(1 - b, g + 1)

                gather_wait(b)

                @pl.when(g >= 2)
                def _():
                    out_wait(b)

                compute(b, g)
                out_start(b, g)
            return carry

        lax.fori_loop(0, n_pairs, pair_body, 0)
        out_wait(0)
        out_wait(1)

    return sc_embed


def kernel(input_ids, token_table, position_table):
    B, L = input_ids.shape
    D1 = token_table.shape[-1]
    N = B * L
    info = plsc.get_sparse_core_info()
    nw = info.num_cores * info.num_subcores
    rows_per_w = N // nw
    ids = input_ids.reshape(N).astype(jnp.int32)
    tok_sp = token_table[:, 1:]
    pos_sp = position_table[:, 1:]
    sc_embed = _build_sc_kernel(
        N, L, D1, position_table.shape[0], rows_per_w, info.num_cores
    )
    out = sc_embed(ids, tok_sp, pos_sp)
    return out.reshape(B, L, D1)


# speedup vs baseline: 2.4417x; 2.2208x over previous
import functools

import jax
import jax.numpy as jnp
from jax import lax
from jax.experimental import pallas as pl
from jax.experimental.pallas import tpu as pltpu
from jax.experimental.pallas import tpu_sc as plsc

_LANES = 16
_CHUNK = 128


def _sqrt_newton(x):
    i = lax.bitcast_convert_type(x, jnp.int32)
    i = jnp.int32(0x5F3759DF) - lax.shift_right_logical(i, 1)
    y = lax.bitcast_convert_type(i, jnp.float32)
    for _ in range(3):
        y = y * (jnp.float32(1.5) - jnp.float32(0.5) * x * y * y)
    return x * y


def _build_sc_kernel(N, L, D1, MAXPOS, rows_per_w, nc):
    mesh = plsc.VectorSubcoreMesh(core_axis_name="c", subcore_axis_name="s")
    n_chunks = rows_per_w // _CHUNK
    n_pairs = n_chunks // 2
    D = D1 - 1

    @functools.partial(
        pl.kernel,
        mesh=mesh,
        compiler_params=pltpu.CompilerParams(
            needs_layout_passes=False, use_tc_tiling_on_sc=False
        ),
        out_type=jax.ShapeDtypeStruct((N, D1), jnp.float32),
        scratch_types=[
            pltpu.VMEM((rows_per_w,), jnp.int32),
            pltpu.VMEM((_CHUNK, D), jnp.float32),
            pltpu.VMEM((_CHUNK, D), jnp.float32),
            pltpu.VMEM((_CHUNK, D1), jnp.float32),
            pltpu.VMEM((_CHUNK, D1), jnp.float32),
            pltpu.VMEM((MAXPOS, D), jnp.float32),
            pltpu.SemaphoreType.DMA,
            pltpu.SemaphoreType.DMA,
            pltpu.SemaphoreType.DMA,
            pltpu.SemaphoreType.DMA,
        ],
    )
    def sc_embed(ids_hbm, tok_hbm, pos_hbm, out_hbm,
                 idx_all, row0_v, row1_v, out0_v, out1_v, pos_v,
                 gsem0, gsem1, osem0, osem1):
        wid = lax.axis_index("s") * nc + lax.axis_index("c")
        base = wid * rows_per_w
        pltpu.sync_copy(pos_hbm, pos_v)
        pltpu.sync_copy(ids_hbm.at[pl.ds(base, rows_per_w)], idx_all)
        lanes = lax.iota(jnp.int32, _LANES)
        zeros_i = jnp.zeros((_LANES,), jnp.int32)
        rows = (row0_v, row1_v)
        outs = (out0_v, out1_v)
        gsems = (gsem0, gsem1)
        osems = (osem0, osem1)

        def gather_start(b, g):
            pltpu.make_async_copy(
                tok_hbm.at[idx_all.at[pl.ds(g * _CHUNK, _CHUNK)]],
                rows[b], gsems[b]).start()

        def gather_wait(b):
            pltpu.make_async_copy(
                tok_hbm.at[idx_all.at[pl.ds(0, _CHUNK)]],
                rows[b], gsems[b]).wait()

        def out_start(b, g):
            pltpu.make_async_copy(
                outs[b], out_hbm.at[pl.ds(base + g * _CHUNK, _CHUNK)],
                osems[b]).start()

        def out_wait(b):
            pltpu.make_async_copy(
                outs[b], out_hbm.at[pl.ds(base, _CHUNK)], osems[b]).wait()

        def compute(b, g):
            row0 = base + g * _CHUNK

            def grp_body(q, c2):
                rvec = q * _LANES + lanes
                pvec = lax.rem(row0 + rvec, L)
                accs = [None, None, None, None]
                for j in range(D):
                    colv = (lanes + j) & (D - 1)
                    tv = plsc.load_gather(rows[b], [rvec, colv])
                    pv = plsc.load_gather(pos_v, [pvec, colv])
                    sp = tv + pv
                    plsc.store_scatter(outs[b], [rvec, colv + 1], sp)
                    sq = sp * sp
                    k = j % 4
                    accs[k] = sq if accs[k] is None else accs[k] + sq
                acc = (accs[0] + accs[1]) + (accs[2] + accs[3])
                t = _sqrt_newton(acc + jnp.float32(1.0))
                plsc.store_scatter(outs[b], [rvec, zeros_i], t)
                return c2

            lax.fori_loop(0, _CHUNK // _LANES, grp_body, 0)

        gather_start(0, 0)

        def pair_body(h, carry):
            for b in (0, 1):
                g = 2 * h + b

                @pl.when(g + 1 < n_chunks)
                def _():
                    gather_start(1 - b, g + 1)

                gather_wait(b)

                @pl.when(g >= 2)
                def _():
                    out_wait(b)

                compute(b, g)
                out_start(b, g)
            return carry

        lax.fori_loop(0, n_pairs, pair_body, 0)
        out_wait(0)
        out_wait(1)

    return sc_embed


def kernel(input_ids, token_table, position_table):
    B, L = input_ids.shape
    D1 = token_table.shape[-1]
    N = B * L
    info = plsc.get_sparse_core_info()
    nw = info.num_cores * info.num_subcores
    rows_per_w = N // nw
    ids = input_ids.reshape(N).astype(jnp.int32)
    tok_sp = token_table[:, 1:]
    pos_sp = position_table[:, 1:]
    sc_embed = _build_sc_kernel(
        N, L, D1, position_table.shape[0], rows_per_w, info.num_cores
    )
    out = sc_embed(ids, tok_sp, pos_sp)
    return out.reshape(B, L, D1)
